# baseline (speedup 1.0000x reference)
import functools

import jax
import jax.numpy as jnp
from jax import lax
from jax.experimental import pallas as pl
from jax.experimental.pallas import tpu as pltpu
from jax.experimental.pallas import tpu_sc as plsc

_LAMBDA = 1e-05
_B = 16384
_D = 16
_L = 16
_NC = 2
_NS = 16
_NW = _NC * _NS
_BPW = _B // _NW

_NFC = 7812
_CPT = 245
_NPASS = 16
_CPP = 16
_LCAP = 1040
_WCAP = 128
_NSTG = _B + 1
_LEFT = _NFC * 128


def _k1_body(nid_hbm, bid_hbm, emdt_hbm, packed_hbm, posmap_hbm,
             ids_sl, lid_n, lpos_n, lid_b, lpos_b,
             chunk, stage, wl_id, wl_pos, wl_all,
             sem_ch, sem_sc):
    w = lax.axis_index("s") * _NC + lax.axis_index("c")
    c0 = w * _CPT
    c1 = jnp.minimum(c0 + _CPT, _NFC)
    nlo = c0 * 128
    nhi = c1 * 128
    lanes = lax.iota(jnp.int32, _L)

    def bucket(ids_hbm, lid, lpos):
        def outer(s, cur):
            pltpu.sync_copy(ids_hbm.at[pl.ds(s * 2048, 2048)], ids_sl)

            def inner(g, cur):
                i16 = ids_sl[pl.ds(g * 16, 16)]
                m = (i16 >= nlo) & (i16 < nhi)
                cur_c = jnp.minimum(cur, _LCAP - 16)
                plsc.store_compressed(lid.at[pl.ds(cur_c, 16)], i16, mask=m)
                pos16 = s * 2048 + g * 16 + lanes
                plsc.store_compressed(lpos.at[pl.ds(cur_c, 16)], pos16, mask=m)
                cnt = plsc.all_reduce_population_count(m)[0]
                return cur + cnt

            return lax.fori_loop(0, 128, inner, cur)

        return lax.fori_loop(0, 8, outer, 0)

    ne_n = bucket(nid_hbm, lid_n, lpos_n)
    ne_b = bucket(bid_hbm, lid_b, lpos_b)

    def do_pass(p, _):
        pc0 = c0 + p * _CPP

        for cc in range(_CPP):
            @pl.when(pc0 + cc < c1)
            def _fire(cc=cc):
                pltpu.async_copy(
                    emdt_hbm.at[:, pl.ds((pc0 + cc) * 128, 128)],
                    chunk.at[cc], sem_ch)

        def side(lid, lpos, ne, side_idx, wait_chunk):
            def wscan(eg, wcur):
                e16 = eg * 16 + lanes
                i16 = lid[pl.ds(eg * 16, 16)]
                tc = i16 >> 7
                m = (e16 < ne) & (tc >= pc0) & (tc < pc0 + _CPP) & (tc < c1)
                wcur_c = jnp.minimum(wcur, _WCAP - 16)
                plsc.store_compressed(wl_id.at[pl.ds(wcur_c, 16)], i16, mask=m)
                plsc.store_compressed(
                    wl_pos.at[pl.ds(wcur_c, 16)], lpos[pl.ds(eg * 16, 16)],
                    mask=m)
                return wcur + plsc.all_reduce_population_count(m)[0]

            nw = lax.fori_loop(0, (ne + 15) // 16, wscan, 0)

            def pad(wg, _):
                sl = pl.ds(wg * 16, 16)
                e16 = wg * 16 + lanes
                m = e16 >= nw
                dump = _B + w * _WCAP + e16
                wl_pos[sl] = jnp.where(m, dump, wl_pos[sl])
                wl_id[sl] = jnp.where(m, nlo, wl_id[sl])
                return 0

            lax.fori_loop(0, _WCAP // 16, pad, 0)

            if wait_chunk:
                for cc in range(_CPP):
                    @pl.when(pc0 + cc < c1)
                    def _drain(cc=cc):
                        pltpu.make_async_copy(
                            emdt_hbm.at[:, pl.ds((pc0 + cc) * 128, 128)],
                            chunk.at[cc], sem_ch).wait()

            b = p * 2 + side_idx
            boff = b * (_WCAP * _D)

            def gather(wg, _):
                e16 = wg * 16 + lanes
                i16 = wl_id[pl.ds(wg * 16, 16)]
                colq = jnp.clip((i16 >> 7) - pc0, 0, _CPP - 1)
                lane = i16 & 127
                for d in range(_D):
                    dcol = jnp.full((_L,), d, jnp.int32)
                    v = plsc.load_gather(chunk, [colq, dcol, lane])
                    plsc.store_scatter(stage, [boff + e16 * _D + d], v)
                return 0

            lax.fori_loop(0, _WCAP // 16, gather, 0)
            block = w * (_NPASS * 2) + b
            pltpu.async_copy(
                stage.at[pl.ds(boff, _WCAP * _D)],
                packed_hbm.at[pl.ds(block * (_WCAP * _D), _WCAP * _D)],
                sem_sc)

            def wlmv(wg, _):
                sl = pl.ds(wg * 16, 16)
                wl_all[b, sl] = wl_pos[sl]
                return 0

            lax.fori_loop(0, _WCAP // 16, wlmv, 0)

        side(lid_n, lpos_n, ne_n, 0, True)
        side(lid_b, lpos_b, ne_b, 1, False)
        return 0

    lax.fori_loop(0, _NPASS, do_pass, 0)

    def drain(b, _):
        boff = b * (_WCAP * _D)
        block = w * (_NPASS * 2) + b
        pltpu.make_async_copy(
            stage.at[pl.ds(boff, _WCAP * _D)],
            packed_hbm.at[pl.ds(block * (_WCAP * _D), _WCAP * _D)],
            sem_sc).wait()
        return 0

    lax.fori_loop(0, _NPASS * 2, drain, 0)
    pltpu.sync_copy(
        wl_all, posmap_hbm.at[pl.ds(w * (_NPASS * 2), _NPASS * 2), :])


_k1 = functools.partial(
    pl.kernel,
    mesh=plsc.VectorSubcoreMesh(core_axis_name="c", subcore_axis_name="s"),
    compiler_params=pltpu.CompilerParams(
        needs_layout_passes=False, use_tc_tiling_on_sc=True
    ),
    out_type=[
        jax.ShapeDtypeStruct((_NW * _NPASS * 2 * _WCAP * _D,), jnp.float32),
        jax.ShapeDtypeStruct((_NW * _NPASS * 2, _WCAP), jnp.int32),
    ],
    scratch_types=[
        pltpu.VMEM((2048,), jnp.int32),
        pltpu.VMEM((_LCAP,), jnp.int32),
        pltpu.VMEM((_LCAP,), jnp.int32),
        pltpu.VMEM((_LCAP,), jnp.int32),
        pltpu.VMEM((_LCAP,), jnp.int32),
        pltpu.VMEM((_CPP, _D, 128), jnp.float32),
        pltpu.VMEM((_NPASS * 2 * _WCAP * _D,), jnp.float32),
        pltpu.VMEM((_WCAP,), jnp.int32),
        pltpu.VMEM((_WCAP,), jnp.int32),
        pltpu.VMEM((_NPASS * 2, _WCAP), jnp.int32),
        pltpu.SemaphoreType.DMA,
        pltpu.SemaphoreType.DMA,
    ],
)(_k1_body)


_NROWS = _NW * _NPASS * 2 * _WCAP
_NSTG2 = _B + _NW * _WCAP


def _k1b_body(packed2d_hbm, posmap_hbm, rows_n_hbm, rows_b_hbm,
              posv, stage_a, stage_b, sem_ld, sem_sc):
    w = lax.axis_index("s") * _NC + lax.axis_index("c")
    pltpu.sync_copy(posmap_hbm.at[pl.ds(w * _NPASS * 2, _NPASS * 2), :], posv)

    def do_block(b, _):
        bf = w * (_NPASS * 2) + b

        @pl.when(b % 2 == 0)
        def _even():
            pltpu.sync_copy(
                packed2d_hbm.at[pl.ds(bf * _WCAP, _WCAP), :], stage_a)
            pltpu.async_copy(
                stage_a, rows_n_hbm.at[posv.at[b]], sem_sc).wait()

        @pl.when(b % 2 == 1)
        def _odd():
            pltpu.sync_copy(
                packed2d_hbm.at[pl.ds(bf * _WCAP, _WCAP), :], stage_b)
            pltpu.async_copy(
                stage_b, rows_b_hbm.at[posv.at[b]], sem_sc).wait()

        return 0

    lax.fori_loop(0, _NPASS * 2, do_block, 0)


_k1b = functools.partial(
    pl.kernel,
    mesh=plsc.VectorSubcoreMesh(core_axis_name="c", subcore_axis_name="s"),
    compiler_params=pltpu.CompilerParams(
        needs_layout_passes=False, use_tc_tiling_on_sc=False
    ),
    out_type=[
        jax.ShapeDtypeStruct((_NSTG2, _D), jnp.float32),
        jax.ShapeDtypeStruct((_NSTG2, _D), jnp.float32),
    ],
    scratch_types=[
        pltpu.VMEM((_NPASS * 2, _WCAP), jnp.int32),
        pltpu.VMEM((_WCAP, _D), jnp.float32),
        pltpu.VMEM((_WCAP, _D), jnp.float32),
        pltpu.SemaphoreType.DMA,
        pltpu.SemaphoreType.DMA,
    ],
)(_k1b_body)


def _k2_body(rows_n_hbm, rows_b_hbm, nid_hbm, bid_hbm, left_hbm,
             bias_hbm, score_hbm, sq_hbm,
             idn_v, idb_v,
             rown_v, rowb_v, bias_v, left_v, score_v, sq_stage,
             sem_n, sem_b, sem_bias):
    w = lax.axis_index("s") * _NC + lax.axis_index("c")
    base = w * _BPW
    lanes = lax.iota(jnp.int32, _L)

    pltpu.sync_copy(nid_hbm.at[pl.ds(base, _BPW)], idn_v)
    pltpu.sync_copy(bid_hbm.at[pl.ds(base, _BPW)], idb_v)
    cp_n = pltpu.async_copy(rows_n_hbm.at[pl.ds(base, _BPW), :], rown_v, sem_n)
    cp_b = pltpu.async_copy(rows_b_hbm.at[pl.ds(base, _BPW), :], rowb_v, sem_b)
    cp_bias = pltpu.async_copy(bias_hbm.at[idb_v], bias_v, sem_bias)
    pltpu.sync_copy(left_hbm, left_v)
    cp_n.wait()
    cp_b.wait()
    cp_bias.wait()

    def group(g, sq_acc):
        row0 = g * _L
        rows_idx = row0 + lanes
        i_n = idn_v[pl.ds(row0, _L)]
        i_b = idb_v[pl.ds(row0, _L)]
        m_n = i_n >= _LEFT
        m_b = i_b >= _LEFT
        any_left = jnp.any(m_n | m_b)
        off_n = jnp.clip(i_n - _LEFT, 0, 63) * _D
        off_b = jnp.clip(i_b - _LEFT, 0, 63) * _D
        acc = jnp.zeros((_L,), jnp.float32)

        def slow(d, carry):
            acc, sq_acc = carry
            dcol = jnp.full((_L,), d, jnp.int32)
            a = plsc.load_gather(rown_v, [rows_idx, dcol])
            b = plsc.load_gather(rowb_v, [rows_idx, dcol])
            al = plsc.load_gather(left_v, [off_n + d])
            bl = plsc.load_gather(left_v, [off_b + d])
            a = jnp.where(m_n, al, a)
            b = jnp.where(m_b, bl, b)
            return acc + a * b, sq_acc + a * a + b * b

        def fast(d, carry):
            acc, sq_acc = carry
            dcol = jnp.full((_L,), d, jnp.int32)
            a = plsc.load_gather(rown_v, [rows_idx, dcol])
            b = plsc.load_gather(rowb_v, [rows_idx, dcol])
            return acc + a * b, sq_acc + a * a + b * b

        acc, sq_acc = lax.cond(
            any_left,
            lambda c: lax.fori_loop(0, _D, slow, c),
            lambda c: lax.fori_loop(0, _D, fast, c),
            (acc, sq_acc),
        )
        bv = bias_v[pl.ds(row0, _L)]
        sq_acc = sq_acc + bv * bv
        score_v[pl.ds(row0, _L)] = acc + bv
        return sq_acc

    sq_acc = lax.fori_loop(0, _BPW // _L, group,
                           jnp.zeros((_L,), jnp.float32))
    sq_stage[...] = sq_acc

    pltpu.sync_copy(score_v, score_hbm.at[pl.ds(base, _BPW)])
    pltpu.sync_copy(sq_stage, sq_hbm.at[w])


_k2 = functools.partial(
    pl.kernel,
    mesh=plsc.VectorSubcoreMesh(core_axis_name="c", subcore_axis_name="s"),
    compiler_params=pltpu.CompilerParams(
        needs_layout_passes=False, use_tc_tiling_on_sc=False
    ),
    out_type=[
        jax.ShapeDtypeStruct((_B,), jnp.float32),
        jax.ShapeDtypeStruct((_NW, _L), jnp.float32),
    ],
    scratch_types=[
        pltpu.VMEM((_BPW,), jnp.int32),
        pltpu.VMEM((_BPW,), jnp.int32),
        pltpu.VMEM((_BPW, _D), jnp.float32),
        pltpu.VMEM((_BPW, _D), jnp.float32),
        pltpu.VMEM((_BPW,), jnp.float32),
        pltpu.VMEM((64 * _D,), jnp.float32),
        pltpu.VMEM((_BPW,), jnp.float32),
        pltpu.VMEM((_L,), jnp.float32),
        pltpu.SemaphoreType.DMA,
        pltpu.SemaphoreType.DMA,
        pltpu.SemaphoreType.DMA,
    ],
)(_k2_body)


def _tc_loss_body(score_ref, label_ref, sq_ref, out_ref):
    x = score_ref[...]
    y = label_ref[...]
    bce = jnp.maximum(x, 0.0) - x * y + jnp.log1p(jnp.exp(-jnp.abs(x)))
    loss = jnp.sum(bce) / _B + (0.5 * _LAMBDA) * jnp.sum(sq_ref[...])
    out_ref[...] = loss.reshape(1, 1)


def kernel(node_ids, neighbor_ids, label, node_emd, bias_vector):
    emd_t = node_emd.T
    leftover = node_emd[_LEFT:].reshape(-1)
    packed, posmap = _k1(node_ids, neighbor_ids, emd_t)
    packed2d = packed.reshape(_NROWS, _D)
    rows_n, rows_b = _k1b(packed2d, posmap)
    score, sq = _k2(rows_n, rows_b, node_ids, neighbor_ids, leftover,
                    bias_vector)
    loss = pl.pallas_call(
        _tc_loss_body,
        out_shape=jax.ShapeDtypeStruct((1, 1), jnp.float32),
    )(score.reshape(128, 128), label.reshape(128, 128), sq)
    return loss[0, 0]


# speedup vs baseline: 1.0078x; 1.0078x over previous
import functools

import jax
import jax.numpy as jnp
from jax import lax
from jax.experimental import pallas as pl
from jax.experimental.pallas import tpu as pltpu
from jax.experimental.pallas import tpu_sc as plsc

_LAMBDA = 1e-05
_B = 16384
_D = 16
_L = 16
_NC = 2
_NS = 16
_NW = _NC * _NS
_BPW = _B // _NW

_NFC = 7812
_CPT = 245
_NPASS = 16
_CPP = 16
_LCAP = 1040
_WCAP = 128
_NSTG = _B + 1
_LEFT = _NFC * 128


def _k1_body(nid_hbm, bid_hbm, emdt_hbm, packed_hbm, posmap_hbm,
             ids_sl, lid_n, lid_b,
             chunk, stage, wl_id, wl_pos, wl_all,
             sem_ch, sem_sc):
    w = lax.axis_index("s") * _NC + lax.axis_index("c")
    c0 = w * _CPT
    c1 = jnp.minimum(c0 + _CPT, _NFC)
    nlo = c0 * 128
    nhi = c1 * 128
    lanes = lax.iota(jnp.int32, _L)

    def bucket(ids_hbm, lpk):
        def outer(s, cur):
            pltpu.sync_copy(ids_hbm.at[pl.ds(s * 2048, 2048)], ids_sl)

            def inner(g, cur):
                i16 = ids_sl[pl.ds(g * 16, 16)]
                m = (i16 >= nlo) & (i16 < nhi)
                cur_c = jnp.minimum(cur, _LCAP - 16)
                pk = ((i16 - nlo) << 14) | (s * 2048 + g * 16 + lanes)
                plsc.store_compressed(lpk.at[pl.ds(cur_c, 16)], pk, mask=m)
                cnt = plsc.all_reduce_population_count(m)[0]
                return cur + cnt

            return lax.fori_loop(0, 128, inner, cur)

        return lax.fori_loop(0, 8, outer, 0)

    ne_n = bucket(nid_hbm, lid_n)
    ne_b = bucket(bid_hbm, lid_b)

    def do_pass(p, _):
        pc0 = c0 + p * _CPP

        for cc in range(_CPP):
            @pl.when(pc0 + cc < c1)
            def _fire(cc=cc):
                pltpu.async_copy(
                    emdt_hbm.at[:, pl.ds((pc0 + cc) * 128, 128)],
                    chunk.at[cc], sem_ch)

        def side(lid, ne, side_idx, wait_chunk):
            def wscan(eg, wcur):
                e16 = eg * 16 + lanes
                pk = lid[pl.ds(eg * 16, 16)]
                lc = pk >> (14 + 7)
                pc0l = pc0 - c0
                m = (e16 < ne) & (lc >= pc0l) & (lc < pc0l + _CPP)
                wcur_c = jnp.minimum(wcur, _WCAP - 16)
                plsc.store_compressed(wl_id.at[pl.ds(wcur_c, 16)], pk, mask=m)
                return wcur + plsc.all_reduce_population_count(m)[0]

            nw = lax.fori_loop(0, (ne + 15) // 16, wscan, 0)

            def pad(wg, _):
                sl = pl.ds(wg * 16, 16)
                e16 = wg * 16 + lanes
                m = e16 >= nw
                pk = jnp.where(m, 0, wl_id[sl])
                wl_id[sl] = pk
                dump = _B + w * _WCAP + e16
                wl_pos[sl] = jnp.where(m, dump, pk & 16383)
                return 0

            lax.fori_loop(0, _WCAP // 16, pad, 0)

            if wait_chunk:
                for cc in range(_CPP):
                    @pl.when(pc0 + cc < c1)
                    def _drain(cc=cc):
                        pltpu.make_async_copy(
                            emdt_hbm.at[:, pl.ds((pc0 + cc) * 128, 128)],
                            chunk.at[cc], sem_ch).wait()

            b = p * 2 + side_idx
            boff = b * (_WCAP * _D)

            def gather(wg, _):
                e16 = wg * 16 + lanes
                pk = wl_id[pl.ds(wg * 16, 16)]
                colq = jnp.clip((pk >> 21) - (pc0 - c0), 0, _CPP - 1)
                lane = (pk >> 14) & 127
                for d in range(_D):
                    dcol = jnp.full((_L,), d, jnp.int32)
                    v = plsc.load_gather(chunk, [colq, dcol, lane])
                    plsc.store_scatter(stage, [boff + e16 * _D + d], v)
                return 0

            lax.fori_loop(0, _WCAP // 16, gather, 0)
            block = w * (_NPASS * 2) + b
            pltpu.async_copy(
                stage.at[pl.ds(boff, _WCAP * _D)],
                packed_hbm.at[pl.ds(block * (_WCAP * _D), _WCAP * _D)],
                sem_sc)

            def wlmv(wg, _):
                sl = pl.ds(wg * 16, 16)
                wl_all[b, sl] = wl_pos[sl]
                return 0

            lax.fori_loop(0, _WCAP // 16, wlmv, 0)

        side(lid_n, ne_n, 0, True)
        side(lid_b, ne_b, 1, False)
        return 0

    lax.fori_loop(0, _NPASS, do_pass, 0)

    def drain(b, _):
        boff = b * (_WCAP * _D)
        block = w * (_NPASS * 2) + b
        pltpu.make_async_copy(
            stage.at[pl.ds(boff, _WCAP * _D)],
            packed_hbm.at[pl.ds(block * (_WCAP * _D), _WCAP * _D)],
            sem_sc).wait()
        return 0

    lax.fori_loop(0, _NPASS * 2, drain, 0)
    pltpu.sync_copy(
        wl_all, posmap_hbm.at[pl.ds(w * (_NPASS * 2), _NPASS * 2), :])


_k1 = functools.partial(
    pl.kernel,
    mesh=plsc.VectorSubcoreMesh(core_axis_name="c", subcore_axis_name="s"),
    compiler_params=pltpu.CompilerParams(
        needs_layout_passes=False, use_tc_tiling_on_sc=True
    ),
    out_type=[
        jax.ShapeDtypeStruct((_NW * _NPASS * 2 * _WCAP * _D,), jnp.float32),
        jax.ShapeDtypeStruct((_NW * _NPASS * 2, _WCAP), jnp.int32),
    ],
    scratch_types=[
        pltpu.VMEM((2048,), jnp.int32),
        pltpu.VMEM((_LCAP,), jnp.int32),
        pltpu.VMEM((_LCAP,), jnp.int32),
        pltpu.VMEM((_CPP, _D, 128), jnp.float32),
        pltpu.VMEM((_NPASS * 2 * _WCAP * _D,), jnp.float32),
        pltpu.VMEM((_WCAP,), jnp.int32),
        pltpu.VMEM((_WCAP,), jnp.int32),
        pltpu.VMEM((_NPASS * 2, _WCAP), jnp.int32),
        pltpu.SemaphoreType.DMA,
        pltpu.SemaphoreType.DMA,
    ],
)(_k1_body)


_NROWS = _NW * _NPASS * 2 * _WCAP
_NSTG2 = _B + _NW * _WCAP


def _k1b_body(packed2d_hbm, posmap_hbm, rows_n_hbm, rows_b_hbm,
              posv, stage_a, stage_b, sem_ld, sem_sc):
    w = lax.axis_index("s") * _NC + lax.axis_index("c")
    pltpu.sync_copy(posmap_hbm.at[pl.ds(w * _NPASS * 2, _NPASS * 2), :], posv)

    def do_block(b, _):
        bf = w * (_NPASS * 2) + b

        @pl.when(b % 2 == 0)
        def _even():
            pltpu.sync_copy(
                packed2d_hbm.at[pl.ds(bf * _WCAP, _WCAP), :], stage_a)
            pltpu.async_copy(
                stage_a, rows_n_hbm.at[posv.at[b]], sem_sc).wait()

        @pl.when(b % 2 == 1)
        def _odd():
            pltpu.sync_copy(
                packed2d_hbm.at[pl.ds(bf * _WCAP, _WCAP), :], stage_b)
            pltpu.async_copy(
                stage_b, rows_b_hbm.at[posv.at[b]], sem_sc).wait()

        return 0

    lax.fori_loop(0, _NPASS * 2, do_block, 0)


_k1b = functools.partial(
    pl.kernel,
    mesh=plsc.VectorSubcoreMesh(core_axis_name="c", subcore_axis_name="s"),
    compiler_params=pltpu.CompilerParams(
        needs_layout_passes=False, use_tc_tiling_on_sc=False
    ),
    out_type=[
        jax.ShapeDtypeStruct((_NSTG2, _D), jnp.float32),
        jax.ShapeDtypeStruct((_NSTG2, _D), jnp.float32),
    ],
    scratch_types=[
        pltpu.VMEM((_NPASS * 2, _WCAP), jnp.int32),
        pltpu.VMEM((_WCAP, _D), jnp.float32),
        pltpu.VMEM((_WCAP, _D), jnp.float32),
        pltpu.SemaphoreType.DMA,
        pltpu.SemaphoreType.DMA,
    ],
)(_k1b_body)


def _k2_body(rows_n_hbm, rows_b_hbm, nid_hbm, bid_hbm, left_hbm,
             bias_hbm, score_hbm, sq_hbm,
             idn_v, idb_v,
             rown_v, rowb_v, bias_v, left_v, score_v, sq_stage,
             sem_n, sem_b, sem_bias):
    w = lax.axis_index("s") * _NC + lax.axis_index("c")
    base = w * _BPW
    lanes = lax.iota(jnp.int32, _L)

    pltpu.sync_copy(nid_hbm.at[pl.ds(base, _BPW)], idn_v)
    pltpu.sync_copy(bid_hbm.at[pl.ds(base, _BPW)], idb_v)
    cp_n = pltpu.async_copy(rows_n_hbm.at[pl.ds(base, _BPW), :], rown_v, sem_n)
    cp_b = pltpu.async_copy(rows_b_hbm.at[pl.ds(base, _BPW), :], rowb_v, sem_b)
    cp_bias = pltpu.async_copy(bias_hbm.at[idb_v], bias_v, sem_bias)
    pltpu.sync_copy(left_hbm, left_v)
    cp_n.wait()
    cp_b.wait()
    cp_bias.wait()

    def group(g, sq_acc):
        row0 = g * _L
        rows_idx = row0 + lanes
        i_n = idn_v[pl.ds(row0, _L)]
        i_b = idb_v[pl.ds(row0, _L)]
        m_n = i_n >= _LEFT
        m_b = i_b >= _LEFT
        any_left = jnp.any(m_n | m_b)
        off_n = jnp.clip(i_n - _LEFT, 0, 63) * _D
        off_b = jnp.clip(i_b - _LEFT, 0, 63) * _D
        acc = jnp.zeros((_L,), jnp.float32)

        def slow(d, carry):
            acc, sq_acc = carry
            dcol = jnp.full((_L,), d, jnp.int32)
            a = plsc.load_gather(rown_v, [rows_idx, dcol])
            b = plsc.load_gather(rowb_v, [rows_idx, dcol])
            al = plsc.load_gather(left_v, [off_n + d])
            bl = plsc.load_gather(left_v, [off_b + d])
            a = jnp.where(m_n, al, a)
            b = jnp.where(m_b, bl, b)
            return acc + a * b, sq_acc + a * a + b * b

        def fast(d, carry):
            acc, sq_acc = carry
            dcol = jnp.full((_L,), d, jnp.int32)
            a = plsc.load_gather(rown_v, [rows_idx, dcol])
            b = plsc.load_gather(rowb_v, [rows_idx, dcol])
            return acc + a * b, sq_acc + a * a + b * b

        acc, sq_acc = lax.cond(
            any_left,
            lambda c: lax.fori_loop(0, _D, slow, c),
            lambda c: lax.fori_loop(0, _D, fast, c),
            (acc, sq_acc),
        )
        bv = bias_v[pl.ds(row0, _L)]
        sq_acc = sq_acc + bv * bv
        score_v[pl.ds(row0, _L)] = acc + bv
        return sq_acc

    sq_acc = lax.fori_loop(0, _BPW // _L, group,
                           jnp.zeros((_L,), jnp.float32))
    sq_stage[...] = sq_acc

    pltpu.sync_copy(score_v, score_hbm.at[pl.ds(base, _BPW)])
    pltpu.sync_copy(sq_stage, sq_hbm.at[w])


_k2 = functools.partial(
    pl.kernel,
    mesh=plsc.VectorSubcoreMesh(core_axis_name="c", subcore_axis_name="s"),
    compiler_params=pltpu.CompilerParams(
        needs_layout_passes=False, use_tc_tiling_on_sc=False
    ),
    out_type=[
        jax.ShapeDtypeStruct((_B,), jnp.float32),
        jax.ShapeDtypeStruct((_NW, _L), jnp.float32),
    ],
    scratch_types=[
        pltpu.VMEM((_BPW,), jnp.int32),
        pltpu.VMEM((_BPW,), jnp.int32),
        pltpu.VMEM((_BPW, _D), jnp.float32),
        pltpu.VMEM((_BPW, _D), jnp.float32),
        pltpu.VMEM((_BPW,), jnp.float32),
        pltpu.VMEM((64 * _D,), jnp.float32),
        pltpu.VMEM((_BPW,), jnp.float32),
        pltpu.VMEM((_L,), jnp.float32),
        pltpu.SemaphoreType.DMA,
        pltpu.SemaphoreType.DMA,
        pltpu.SemaphoreType.DMA,
    ],
)(_k2_body)


def _tc_loss_body(score_ref, label_ref, sq_ref, out_ref):
    x = score_ref[...]
    y = label_ref[...]
    bce = jnp.maximum(x, 0.0) - x * y + jnp.log1p(jnp.exp(-jnp.abs(x)))
    loss = jnp.sum(bce) / _B + (0.5 * _LAMBDA) * jnp.sum(sq_ref[...])
    out_ref[...] = loss.reshape(1, 1)


def kernel(node_ids, neighbor_ids, label, node_emd, bias_vector):
    emd_t = node_emd.T
    leftover = node_emd[_LEFT:].reshape(-1)
    packed, posmap = _k1(node_ids, neighbor_ids, emd_t)
    packed2d = packed.reshape(_NROWS, _D)
    rows_n, rows_b = _k1b(packed2d, posmap)
    score, sq = _k2(rows_n, rows_b, node_ids, neighbor_ids, leftover,
                    bias_vector)
    loss = pl.pallas_call(
        _tc_loss_body,
        out_shape=jax.ShapeDtypeStruct((1, 1), jnp.float32),
    )(score.reshape(128, 128), label.reshape(128, 128), sq)
    return loss[0, 0]
